# baseline (speedup 1.0000x reference)
import functools

import jax
import jax.numpy as jnp
from jax import lax
from jax.experimental import pallas as pl
from jax.experimental.pallas import tpu as pltpu
from jax.experimental.pallas import tpu_sc as plsc

V = 1000000
D = 64
NB = 4096
NS = 200
LANES = 16

FULL_BLOCKS = V // 128
TAIL_COLS = V - FULL_BLOCKS * 128
K1_ITERS = (FULL_BLOCKS + 31) // 32
TP_ROWS = (FULL_BLOCKS + 1) * 64


def _iota16():
    return lax.iota(jnp.int32, LANES)


def _build_k1():
    info = plsc.get_sparse_core_info()
    nc = info.num_cores
    mesh = plsc.VectorSubcoreMesh(core_axis_name="c", subcore_axis_name="s")

    scratch = [
        pltpu.VMEM((64, 128), jnp.float32),
        pltpu.VMEM((64, 128), jnp.float32),
        pltpu.VMEM((64, 128), jnp.float32),
        pltpu.VMEM((64, 128), jnp.float32),
        pltpu.VMEM((64, 64), jnp.float32),
        pltpu.VMEM((64, 128), jnp.float32),
        pltpu.SemaphoreType.DMA,
        pltpu.SemaphoreType.DMA,
        pltpu.SemaphoreType.DMA,
        pltpu.SemaphoreType.DMA,
    ]

    @functools.partial(
        pl.kernel,
        mesh=mesh,
        compiler_params=pltpu.CompilerParams(use_tc_tiling_on_sc=True, needs_layout_passes=False, disable_bounds_checks=True),
        out_type=jax.ShapeDtypeStruct((TP_ROWS, 128), jnp.float32),
        scratch_types=scratch,
    )
    def k1(tt, tp, ib0, ib1, ob0, ob1, tib, tob, rs0, rs1, ws0, ws1):
        wid = lax.axis_index("s") * nc + lax.axis_index("c")
        ibufs = (ib0, ib1)
        obufs = (ob0, ob1)
        rsems = (rs0, rs1)
        wsems = (ws0, ws1)
        iota = _iota16()
        skews = [jnp.bitwise_and(iota + kk2, 15) for kk2 in range(16)]

        def valid(k):
            return (wid + 32 * k) < FULL_BLOCKS

        def _rargs(k, slot):
            j = wid + 32 * k
            return (tt.at[:, pl.ds(j * 128, 128)], ibufs[slot], rsems[slot])

        def _wargs(k, slot):
            j = wid + 32 * k
            return (obufs[slot], tp.at[pl.ds(j * 64, 64)], wsems[slot])

        def read(k, slot):
            pltpu.async_copy(*_rargs(k, slot))

        def wait_read(k, slot):
            pltpu.make_async_copy(*_rargs(k, slot)).wait()

        def write(k, slot):
            pltpu.async_copy(*_wargs(k, slot))

        def wait_write(k, slot):
            pltpu.make_async_copy(*_wargs(k, slot)).wait()

        @pl.when(valid(0))
        def _():
            read(0, 0)

        def step(k, slot):
            @pl.when(valid(k))
            def _():
                @pl.when(valid(k + 1))
                def _():
                    read(k + 1, 1 - slot)

                wait_read(k, slot)

                @pl.when(k >= 2)
                def _():
                    wait_write(k - 2, slot)

                ib = ibufs[slot]
                ob = obufs[slot]

                def block16(bb):
                    q0s = jnp.bitwise_and(bb, 3) * 16
                    c0s = jnp.right_shift(bb, 2) * 16
                    h64 = jnp.bitwise_and(c0s, 64)
                    d0 = jnp.bitwise_and(c0s, 63)
                    lcols = iota + (h64 + q0s)
                    srows = iota + q0s
                    for g0 in range(0, 16, 8):
                        vs = [
                            plsc.load_gather(ib, [skews[g0 + i] + d0, lcols])
                            for i in range(8)
                        ]
                        for i in range(8):
                            plsc.store_scatter(
                                ob, [srows, skews[g0 + i] + c0s], vs[i]
                            )

                def bbody(bb2, carry):
                    block16(2 * bb2)
                    block16(2 * bb2 + 1)
                    return carry

                lax.fori_loop(0, 16, bbody, 0)
                write(k, slot)

        def pair(kk, carry):
            step(2 * kk, 0)
            step(2 * kk + 1, 1)
            return carry

        lax.fori_loop(0, (K1_ITERS + 1) // 2, pair, 0)

        @pl.when(valid(K1_ITERS - 2))
        def _():
            wait_write(K1_ITERS - 2, (K1_ITERS - 2) % 2)

        @pl.when(valid(K1_ITERS - 1))
        def _():
            wait_write(K1_ITERS - 1, (K1_ITERS - 1) % 2)

        @pl.when(jnp.logical_and(jnp.logical_not(valid(K1_ITERS - 1)),
                                 valid(K1_ITERS - 3)))
        def _():
            wait_write(K1_ITERS - 3, (K1_ITERS - 3) % 2)

        @pl.when(wid == 0)
        def _():
            pltpu.sync_copy(tt.at[:, pl.ds(FULL_BLOCKS * 128, TAIL_COLS)], tib)

            def tqbody(q, carry):
                for g in range(4):
                    col = jnp.full((LANES,), q, jnp.int32)
                    v = plsc.load_gather(tib, [iota + 16 * g, col])
                    tob[q, pl.ds(16 * g, 16)] = v
                return carry

            lax.fori_loop(0, 64, tqbody, 0)
            pltpu.sync_copy(tob, tp.at[pl.ds(FULL_BLOCKS * 64, 64)])

    return k1


def _build_kg():
    info = plsc.get_sparse_core_info()
    nc = info.num_cores
    mesh = plsc.VectorSubcoreMesh(core_axis_name="c", subcore_axis_name="s")
    GB = 4
    OB = 2

    scratch = (
        [pltpu.VMEM((NS, 128), jnp.int32)]
        + [pltpu.VMEM((GB, 128), jnp.int32)]
        + [pltpu.VMEM((128,), jnp.int32)]
        + [pltpu.VMEM((128, 128), jnp.float32) for _ in range(GB)]
        + [pltpu.VMEM((64, 128), jnp.float32) for _ in range(OB)]
        + [pltpu.SemaphoreType.DMA for _ in range(GB)]
        + [pltpu.SemaphoreType.DMA for _ in range(OB)]
    )

    @functools.partial(
        pl.kernel,
        mesh=mesh,
        compiler_params=pltpu.CompilerParams(use_tc_tiling_on_sc=True, needs_layout_passes=False, disable_bounds_checks=True),
        out_type=jax.ShapeDtypeStruct((NS, D, NB), jnp.float32),
        scratch_types=scratch,
    )
    def kg(tp, it, out, iv, pidx, hbv, *rest):
        gbufs = rest[:GB]
        obufs = rest[GB:GB + OB]
        gsems = rest[GB + OB:GB + OB + GB]
        wsems = rest[GB + OB + GB:]
        wid = lax.axis_index("s") * nc + lax.axis_index("c")
        b0 = wid * 128
        iota = _iota16()
        skews = [jnp.bitwise_and(iota + kk2, 15) for kk2 in range(16)]

        pltpu.sync_copy(it.at[:, pl.ds(b0, 128)], iv)

        def issue_gather(s, slot):
            for rg in range(8):
                vals = iv[s, pl.ds(16 * rg, 16)]
                pidx[slot, pl.ds(16 * rg, 16)] = (
                    jnp.left_shift(jnp.right_shift(vals, 7), 6)
                    + jnp.bitwise_and(vals, 63)
                )
            return pltpu.async_copy(
                tp.at[pidx.at[slot]], gbufs[slot], gsems[slot]
            )

        def _wargs(s, slot):
            return (obufs[slot], out.at[s, :, pl.ds(b0, 128)], wsems[slot])

        def issue_write(s, slot):
            pltpu.async_copy(*_wargs(s, slot))

        def wait_write(s, slot):
            pltpu.make_async_copy(*_wargs(s, slot)).wait()

        for b in range(GB):
            issue_gather(b, b)

        def step(s, slot):
            pltpu.make_async_copy(
                tp.at[pidx.at[slot]], gbufs[slot], gsems[slot]
            ).wait()

            oslot = slot % OB

            @pl.when(s >= OB)
            def _():
                wait_write(s - OB, oslot)

            gb = gbufs[slot]
            ob = obufs[oslot]
            for rg in range(8):
                vals = iv[s, pl.ds(16 * rg, 16)]
                hbv[pl.ds(16 * rg, 16)] = jnp.bitwise_and(vals, 64)

            def rgbody(rg, carry):
                hsub = hbv[pl.ds(rg * 16, 16)]
                rvec = iota + rg * 16
                for d0 in range(0, D, 16):
                    hd = hsub + d0
                    for g0 in range(0, 16, 8):
                        vs = [
                            plsc.load_gather(gb, [rvec, hd + skews[g0 + i]])
                            for i in range(8)
                        ]
                        for i in range(8):
                            plsc.store_scatter(
                                ob, [skews[g0 + i] + d0, rvec], vs[i]
                            )
                return carry

            lax.fori_loop(0, 8, rgbody, 0)
            issue_write(s, oslot)

            @pl.when(s + GB < NS)
            def _():
                issue_gather(s + GB, slot)

        def group(ss, carry):
            for b in range(GB):
                step(ss * GB + b, b)
            return carry

        lax.fori_loop(0, NS // GB, group, 0)

        wait_write(NS - 2, (NS - 2) % GB % OB)
        wait_write(NS - 1, (NS - 1) % GB % OB)

    return kg


def kernel(level_idx, embedding_table):
    tT = embedding_table.T
    idxT = level_idx.astype(jnp.int32).T
    tpair = _build_k1()(tT)
    out3 = _build_kg()(tpair, idxT)
    return out3.transpose(2, 0, 1)


# speedup vs baseline: 1.2469x; 1.2469x over previous
import functools

import jax
import jax.numpy as jnp
from jax import lax
from jax.experimental import pallas as pl
from jax.experimental.pallas import tpu as pltpu
from jax.experimental.pallas import tpu_sc as plsc

V = 1000000
D = 64
NB = 4096
NS = 200
LANES = 16

FULL_BLOCKS = V // 128
TAIL_COLS = V - FULL_BLOCKS * 128
K1_ITERS = (FULL_BLOCKS + 31) // 32
TP_ROWS = (FULL_BLOCKS + 1) * 64


def _iota16():
    return lax.iota(jnp.int32, LANES)


def _build_k1():
    info = plsc.get_sparse_core_info()
    nc = info.num_cores
    mesh = plsc.VectorSubcoreMesh(core_axis_name="c", subcore_axis_name="s")

    scratch = [
        pltpu.VMEM((64, 128), jnp.float32),
        pltpu.VMEM((64, 128), jnp.float32),
        pltpu.VMEM((64, 128), jnp.float32),
        pltpu.VMEM((64, 128), jnp.float32),
        pltpu.VMEM((64, 64), jnp.float32),
        pltpu.VMEM((64, 128), jnp.float32),
        pltpu.SemaphoreType.DMA,
        pltpu.SemaphoreType.DMA,
        pltpu.SemaphoreType.DMA,
        pltpu.SemaphoreType.DMA,
    ]

    @functools.partial(
        pl.kernel,
        mesh=mesh,
        compiler_params=pltpu.CompilerParams(use_tc_tiling_on_sc=True, needs_layout_passes=False, disable_bounds_checks=True),
        out_type=jax.ShapeDtypeStruct((TP_ROWS, 128), jnp.float32),
        scratch_types=scratch,
    )
    def k1(tt, tp, ib0, ib1, ob0, ob1, tib, tob, rs0, rs1, ws0, ws1):
        wid = lax.axis_index("s") * nc + lax.axis_index("c")
        ibufs = (ib0, ib1)
        obufs = (ob0, ob1)
        rsems = (rs0, rs1)
        wsems = (ws0, ws1)
        iota = _iota16()
        skews = [jnp.bitwise_and(iota + kk2, 15) for kk2 in range(16)]

        def valid(k):
            return (wid + 32 * k) < FULL_BLOCKS

        def _rargs(k, slot):
            j = wid + 32 * k
            return (tt.at[:, pl.ds(j * 128, 128)], ibufs[slot], rsems[slot])

        def _wargs(k, slot):
            j = wid + 32 * k
            return (obufs[slot], tp.at[pl.ds(j * 64, 64)], wsems[slot])

        def read(k, slot):
            pltpu.async_copy(*_rargs(k, slot))

        def wait_read(k, slot):
            pltpu.make_async_copy(*_rargs(k, slot)).wait()

        def write(k, slot):
            pltpu.async_copy(*_wargs(k, slot))

        def wait_write(k, slot):
            pltpu.make_async_copy(*_wargs(k, slot)).wait()

        @pl.when(valid(0))
        def _():
            read(0, 0)

        def step(k, slot):
            @pl.when(valid(k))
            def _():
                @pl.when(valid(k + 1))
                def _():
                    read(k + 1, 1 - slot)

                wait_read(k, slot)

                @pl.when(k >= 2)
                def _():
                    wait_write(k - 2, slot)

                ib = ibufs[slot]
                ob = obufs[slot]

                @plsc.parallel_loop(0, 32, unroll=2)
                def bbody(bb):
                    q0s = jnp.bitwise_and(bb, 3) * 16
                    c0s = jnp.right_shift(bb, 2) * 16
                    h64 = jnp.bitwise_and(c0s, 64)
                    d0 = jnp.bitwise_and(c0s, 63)
                    lcols = iota + (h64 + q0s)
                    srows = iota + q0s
                    for g0 in range(0, 16, 8):
                        vs = [
                            plsc.load_gather(ib, [skews[g0 + i] + d0, lcols])
                            for i in range(8)
                        ]
                        for i in range(8):
                            plsc.store_scatter(
                                ob, [srows, skews[g0 + i] + c0s], vs[i]
                            )

                write(k, slot)

        def pair(kk, carry):
            step(2 * kk, 0)
            step(2 * kk + 1, 1)
            return carry

        lax.fori_loop(0, (K1_ITERS + 1) // 2, pair, 0)

        @pl.when(valid(K1_ITERS - 2))
        def _():
            wait_write(K1_ITERS - 2, (K1_ITERS - 2) % 2)

        @pl.when(valid(K1_ITERS - 1))
        def _():
            wait_write(K1_ITERS - 1, (K1_ITERS - 1) % 2)

        @pl.when(jnp.logical_and(jnp.logical_not(valid(K1_ITERS - 1)),
                                 valid(K1_ITERS - 3)))
        def _():
            wait_write(K1_ITERS - 3, (K1_ITERS - 3) % 2)

        @pl.when(wid == 0)
        def _():
            pltpu.sync_copy(tt.at[:, pl.ds(FULL_BLOCKS * 128, TAIL_COLS)], tib)

            def tqbody(q, carry):
                for g in range(4):
                    col = jnp.full((LANES,), q, jnp.int32)
                    v = plsc.load_gather(tib, [iota + 16 * g, col])
                    tob[q, pl.ds(16 * g, 16)] = v
                return carry

            lax.fori_loop(0, 64, tqbody, 0)
            pltpu.sync_copy(tob, tp.at[pl.ds(FULL_BLOCKS * 64, 64)])

    return k1


def _build_kg():
    info = plsc.get_sparse_core_info()
    nc = info.num_cores
    mesh = plsc.VectorSubcoreMesh(core_axis_name="c", subcore_axis_name="s")
    GB = 4
    OB = 2

    scratch = (
        [pltpu.VMEM((NS, 128), jnp.int32)]
        + [pltpu.VMEM((GB, 128), jnp.int32)]
        + [pltpu.VMEM((128,), jnp.int32)]
        + [pltpu.VMEM((128, 128), jnp.float32) for _ in range(GB)]
        + [pltpu.VMEM((64, 128), jnp.float32) for _ in range(OB)]
        + [pltpu.SemaphoreType.DMA for _ in range(GB)]
        + [pltpu.SemaphoreType.DMA for _ in range(OB)]
    )

    @functools.partial(
        pl.kernel,
        mesh=mesh,
        compiler_params=pltpu.CompilerParams(use_tc_tiling_on_sc=True, needs_layout_passes=False, disable_bounds_checks=True),
        out_type=jax.ShapeDtypeStruct((NS, D, NB), jnp.float32),
        scratch_types=scratch,
    )
    def kg(tp, it, out, iv, pidx, hbv, *rest):
        gbufs = rest[:GB]
        obufs = rest[GB:GB + OB]
        gsems = rest[GB + OB:GB + OB + GB]
        wsems = rest[GB + OB + GB:]
        wid = lax.axis_index("s") * nc + lax.axis_index("c")
        b0 = wid * 128
        iota = _iota16()
        skews = [jnp.bitwise_and(iota + kk2, 15) for kk2 in range(16)]

        pltpu.sync_copy(it.at[:, pl.ds(b0, 128)], iv)

        def issue_gather(s, slot):
            for rg in range(8):
                vals = iv[s, pl.ds(16 * rg, 16)]
                pidx[slot, pl.ds(16 * rg, 16)] = (
                    jnp.left_shift(jnp.right_shift(vals, 7), 6)
                    + jnp.bitwise_and(vals, 63)
                )
            return pltpu.async_copy(
                tp.at[pidx.at[slot]], gbufs[slot], gsems[slot]
            )

        def _wargs(s, slot):
            return (obufs[slot], out.at[s, :, pl.ds(b0, 128)], wsems[slot])

        def issue_write(s, slot):
            pltpu.async_copy(*_wargs(s, slot))

        def wait_write(s, slot):
            pltpu.make_async_copy(*_wargs(s, slot)).wait()

        for b in range(GB):
            issue_gather(b, b)

        def step(s, slot):
            pltpu.make_async_copy(
                tp.at[pidx.at[slot]], gbufs[slot], gsems[slot]
            ).wait()

            oslot = slot % OB

            @pl.when(s >= OB)
            def _():
                wait_write(s - OB, oslot)

            gb = gbufs[slot]
            ob = obufs[oslot]
            for rg in range(8):
                vals = iv[s, pl.ds(16 * rg, 16)]
                hbv[pl.ds(16 * rg, 16)] = jnp.bitwise_and(vals, 64)

            @plsc.parallel_loop(0, 8, unroll=2)
            def rgbody(rg):
                hsub = hbv[pl.ds(rg * 16, 16)]
                rvec = iota + rg * 16
                for d0 in range(0, D, 16):
                    hd = hsub + d0
                    for g0 in range(0, 16, 8):
                        vs = [
                            plsc.load_gather(gb, [rvec, hd + skews[g0 + i]])
                            for i in range(8)
                        ]
                        for i in range(8):
                            plsc.store_scatter(
                                ob, [skews[g0 + i] + d0, rvec], vs[i]
                            )

            issue_write(s, oslot)

            @pl.when(s + GB < NS)
            def _():
                issue_gather(s + GB, slot)

        def group(ss, carry):
            for b in range(GB):
                step(ss * GB + b, b)
            return carry

        lax.fori_loop(0, NS // GB, group, 0)

        wait_write(NS - 2, (NS - 2) % GB % OB)
        wait_write(NS - 1, (NS - 1) % GB % OB)

    return kg


def kernel(level_idx, embedding_table):
    tT = embedding_table.T
    idxT = level_idx.astype(jnp.int32).T
    tpair = _build_k1()(tT)
    out3 = _build_kg()(tpair, idxT)
    return out3.transpose(2, 0, 1)


# speedup vs baseline: 1.5832x; 1.2697x over previous
import functools

import jax
import jax.numpy as jnp
from jax import lax
from jax.experimental import pallas as pl
from jax.experimental.pallas import tpu as pltpu
from jax.experimental.pallas import tpu_sc as plsc

V = 1000000
D = 64
NB = 4096
NS = 200
LANES = 16

FULL_BLOCKS = V // 128
TAIL_COLS = V - FULL_BLOCKS * 128
K1_ITERS = (FULL_BLOCKS + 31) // 32
TP_ROWS = (FULL_BLOCKS + 1) * 64


def _iota16():
    return lax.iota(jnp.int32, LANES)


def _build_k1():
    info = plsc.get_sparse_core_info()
    nc = info.num_cores
    mesh = plsc.VectorSubcoreMesh(core_axis_name="c", subcore_axis_name="s")

    scratch = [
        pltpu.VMEM((64, 128), jnp.float32),
        pltpu.VMEM((64, 128), jnp.float32),
        pltpu.VMEM((64, 128), jnp.float32),
        pltpu.VMEM((64, 128), jnp.float32),
        pltpu.VMEM((64, 64), jnp.float32),
        pltpu.VMEM((64, 128), jnp.float32),
        pltpu.SemaphoreType.DMA,
        pltpu.SemaphoreType.DMA,
        pltpu.SemaphoreType.DMA,
        pltpu.SemaphoreType.DMA,
    ]

    @functools.partial(
        pl.kernel,
        mesh=mesh,
        compiler_params=pltpu.CompilerParams(use_tc_tiling_on_sc=True, needs_layout_passes=False, disable_bounds_checks=True),
        out_type=jax.ShapeDtypeStruct((TP_ROWS, 128), jnp.float32),
        scratch_types=scratch,
    )
    def k1(tt, tp, ib0, ib1, ob0, ob1, tib, tob, rs0, rs1, ws0, ws1):
        wid = lax.axis_index("s") * nc + lax.axis_index("c")
        ibufs = (ib0, ib1)
        obufs = (ob0, ob1)
        rsems = (rs0, rs1)
        wsems = (ws0, ws1)
        iota = _iota16()
        skews = [jnp.bitwise_and(iota + kk2, 15) for kk2 in range(16)]

        def valid(k):
            return (wid + 32 * k) < FULL_BLOCKS

        def _rargs(k, slot):
            j = wid + 32 * k
            return (tt.at[:, pl.ds(j * 128, 128)], ibufs[slot], rsems[slot])

        def _wargs(k, slot):
            j = wid + 32 * k
            return (obufs[slot], tp.at[pl.ds(j * 64, 64)], wsems[slot])

        def read(k, slot):
            pltpu.async_copy(*_rargs(k, slot))

        def wait_read(k, slot):
            pltpu.make_async_copy(*_rargs(k, slot)).wait()

        def write(k, slot):
            pltpu.async_copy(*_wargs(k, slot))

        def wait_write(k, slot):
            pltpu.make_async_copy(*_wargs(k, slot)).wait()

        @pl.when(valid(0))
        def _():
            read(0, 0)

        def step(k, slot):
            @pl.when(valid(k))
            def _():
                @pl.when(valid(k + 1))
                def _():
                    read(k + 1, 1 - slot)

                wait_read(k, slot)

                @pl.when(k >= 2)
                def _():
                    wait_write(k - 2, slot)

                ib = ibufs[slot]
                ob = obufs[slot]

                @plsc.parallel_loop(0, 32, unroll=1)
                def bbody(bb):
                    q0s = jnp.bitwise_and(bb, 3) * 16
                    c0s = jnp.right_shift(bb, 2) * 16
                    h64 = jnp.bitwise_and(c0s, 64)
                    d0 = jnp.bitwise_and(c0s, 63)
                    lcols = iota + (h64 + q0s)
                    srows = iota + q0s
                    for g0 in range(0, 16, 8):
                        vs = [
                            plsc.load_gather(ib, [skews[g0 + i] + d0, lcols])
                            for i in range(8)
                        ]
                        for i in range(8):
                            plsc.store_scatter(
                                ob, [srows, skews[g0 + i] + c0s], vs[i]
                            )

                write(k, slot)

        def pair(kk, carry):
            step(2 * kk, 0)
            step(2 * kk + 1, 1)
            return carry

        lax.fori_loop(0, (K1_ITERS + 1) // 2, pair, 0)

        @pl.when(valid(K1_ITERS - 2))
        def _():
            wait_write(K1_ITERS - 2, (K1_ITERS - 2) % 2)

        @pl.when(valid(K1_ITERS - 1))
        def _():
            wait_write(K1_ITERS - 1, (K1_ITERS - 1) % 2)

        @pl.when(jnp.logical_and(jnp.logical_not(valid(K1_ITERS - 1)),
                                 valid(K1_ITERS - 3)))
        def _():
            wait_write(K1_ITERS - 3, (K1_ITERS - 3) % 2)

        @pl.when(wid == 0)
        def _():
            pltpu.sync_copy(tt.at[:, pl.ds(FULL_BLOCKS * 128, TAIL_COLS)], tib)

            def tqbody(q, carry):
                for g in range(4):
                    col = jnp.full((LANES,), q, jnp.int32)
                    v = plsc.load_gather(tib, [iota + 16 * g, col])
                    tob[q, pl.ds(16 * g, 16)] = v
                return carry

            lax.fori_loop(0, 64, tqbody, 0)
            pltpu.sync_copy(tob, tp.at[pl.ds(FULL_BLOCKS * 64, 64)])

    return k1


def _build_kg():
    info = plsc.get_sparse_core_info()
    nc = info.num_cores
    mesh = plsc.VectorSubcoreMesh(core_axis_name="c", subcore_axis_name="s")
    GB = 4
    OB = 2

    scratch = (
        [pltpu.VMEM((NS, 128), jnp.int32)]
        + [pltpu.VMEM((GB, 128), jnp.int32)]
        + [pltpu.VMEM((128,), jnp.int32)]
        + [pltpu.VMEM((128, 128), jnp.float32) for _ in range(GB)]
        + [pltpu.VMEM((64, 128), jnp.float32) for _ in range(OB)]
        + [pltpu.SemaphoreType.DMA for _ in range(GB)]
        + [pltpu.SemaphoreType.DMA for _ in range(OB)]
    )

    @functools.partial(
        pl.kernel,
        mesh=mesh,
        compiler_params=pltpu.CompilerParams(use_tc_tiling_on_sc=True, needs_layout_passes=False, disable_bounds_checks=True),
        out_type=jax.ShapeDtypeStruct((NS, D, NB), jnp.float32),
        scratch_types=scratch,
    )
    def kg(tp, it, out, iv, pidx, hbv, *rest):
        gbufs = rest[:GB]
        obufs = rest[GB:GB + OB]
        gsems = rest[GB + OB:GB + OB + GB]
        wsems = rest[GB + OB + GB:]
        wid = lax.axis_index("s") * nc + lax.axis_index("c")
        b0 = wid * 128
        iota = _iota16()
        skews = [jnp.bitwise_and(iota + kk2, 15) for kk2 in range(16)]

        pltpu.sync_copy(it.at[:, pl.ds(b0, 128)], iv)

        def issue_gather(s, slot):
            for rg in range(8):
                vals = iv[s, pl.ds(16 * rg, 16)]
                pidx[slot, pl.ds(16 * rg, 16)] = (
                    jnp.left_shift(jnp.right_shift(vals, 7), 6)
                    + jnp.bitwise_and(vals, 63)
                )
            return pltpu.async_copy(
                tp.at[pidx.at[slot]], gbufs[slot], gsems[slot]
            )

        def _wargs(s, slot):
            return (obufs[slot], out.at[s, :, pl.ds(b0, 128)], wsems[slot])

        def issue_write(s, slot):
            pltpu.async_copy(*_wargs(s, slot))

        def wait_write(s, slot):
            pltpu.make_async_copy(*_wargs(s, slot)).wait()

        for b in range(GB):
            issue_gather(b, b)

        def step(s, slot):
            pltpu.make_async_copy(
                tp.at[pidx.at[slot]], gbufs[slot], gsems[slot]
            ).wait()

            oslot = slot % OB

            @pl.when(s >= OB)
            def _():
                wait_write(s - OB, oslot)

            gb = gbufs[slot]
            ob = obufs[oslot]
            for rg in range(8):
                vals = iv[s, pl.ds(16 * rg, 16)]
                hbv[pl.ds(16 * rg, 16)] = jnp.bitwise_and(vals, 64)

            @plsc.parallel_loop(0, 8, unroll=1)
            def rgbody(rg):
                hsub = hbv[pl.ds(rg * 16, 16)]
                rvec = iota + rg * 16
                for d0 in range(0, D, 16):
                    hd = hsub + d0
                    for g0 in range(0, 16, 8):
                        vs = [
                            plsc.load_gather(gb, [rvec, hd + skews[g0 + i]])
                            for i in range(8)
                        ]
                        for i in range(8):
                            plsc.store_scatter(
                                ob, [skews[g0 + i] + d0, rvec], vs[i]
                            )

            issue_write(s, oslot)

            @pl.when(s + GB < NS)
            def _():
                issue_gather(s + GB, slot)

        def group(ss, carry):
            for b in range(GB):
                step(ss * GB + b, b)
            return carry

        lax.fori_loop(0, NS // GB, group, 0)

        wait_write(NS - 2, (NS - 2) % GB % OB)
        wait_write(NS - 1, (NS - 1) % GB % OB)

    return kg


def kernel(level_idx, embedding_table):
    tT = embedding_table.T
    idxT = level_idx.astype(jnp.int32).T
    tpair = _build_k1()(tT)
    out3 = _build_kg()(tpair, idxT)
    return out3.transpose(2, 0, 1)


# speedup vs baseline: 1.9262x; 1.2167x over previous
import functools

import jax
import jax.numpy as jnp
from jax import lax
from jax.experimental import pallas as pl
from jax.experimental.pallas import tpu as pltpu
from jax.experimental.pallas import tpu_sc as plsc

V = 1000000
D = 64
NB = 4096
NS = 200
LANES = 16

FULL_BLOCKS = V // 128
TAIL_COLS = V - FULL_BLOCKS * 128
K1_ITERS = (FULL_BLOCKS + 31) // 32
TP_ROWS = (FULL_BLOCKS + 1) * 64


def _iota16():
    return lax.iota(jnp.int32, LANES)


def _build_k1():
    info = plsc.get_sparse_core_info()
    nc = info.num_cores
    mesh = plsc.VectorSubcoreMesh(core_axis_name="c", subcore_axis_name="s")

    scratch = [
        pltpu.VMEM((64, 128), jnp.float32),
        pltpu.VMEM((64, 128), jnp.float32),
        pltpu.VMEM((64, 128), jnp.float32),
        pltpu.VMEM((64, 128), jnp.float32),
        pltpu.VMEM((64, 64), jnp.float32),
        pltpu.VMEM((64, 128), jnp.float32),
        pltpu.SemaphoreType.DMA,
        pltpu.SemaphoreType.DMA,
        pltpu.SemaphoreType.DMA,
        pltpu.SemaphoreType.DMA,
    ]

    @functools.partial(
        pl.kernel,
        mesh=mesh,
        compiler_params=pltpu.CompilerParams(use_tc_tiling_on_sc=True, needs_layout_passes=False, disable_bounds_checks=True),
        out_type=jax.ShapeDtypeStruct((TP_ROWS, 128), jnp.float32),
        scratch_types=scratch,
    )
    def k1(tt, tp, ib0, ib1, ob0, ob1, tib, tob, rs0, rs1, ws0, ws1):
        wid = lax.axis_index("s") * nc + lax.axis_index("c")
        ibufs = (ib0, ib1)
        obufs = (ob0, ob1)
        rsems = (rs0, rs1)
        wsems = (ws0, ws1)
        iota = _iota16()
        skews = [jnp.bitwise_and(iota + kk2, 15) for kk2 in range(16)]

        def valid(k):
            return (wid + 32 * k) < FULL_BLOCKS

        def _rargs(k, slot):
            j = wid + 32 * k
            return (tt.at[:, pl.ds(j * 128, 128)], ibufs[slot], rsems[slot])

        def _wargs(k, slot):
            j = wid + 32 * k
            return (obufs[slot], tp.at[pl.ds(j * 64, 64)], wsems[slot])

        def read(k, slot):
            pltpu.async_copy(*_rargs(k, slot))

        def wait_read(k, slot):
            pltpu.make_async_copy(*_rargs(k, slot)).wait()

        def write(k, slot):
            pltpu.async_copy(*_wargs(k, slot))

        def wait_write(k, slot):
            pltpu.make_async_copy(*_wargs(k, slot)).wait()

        @pl.when(valid(0))
        def _():
            read(0, 0)

        def step(k, slot):
            @pl.when(valid(k))
            def _():
                @pl.when(valid(k + 1))
                def _():
                    read(k + 1, 1 - slot)

                wait_read(k, slot)

                @pl.when(k >= 2)
                def _():
                    wait_write(k - 2, slot)

                ib = ibufs[slot]
                ob = obufs[slot]

                def bbody(bb, carry):
                    q0s = jnp.bitwise_and(bb, 3) * 16
                    c0s = jnp.right_shift(bb, 2) * 16
                    h64 = jnp.bitwise_and(c0s, 64)
                    d0 = jnp.bitwise_and(c0s, 63)
                    lcols = iota + (h64 + q0s)
                    srows = iota + q0s
                    vs = [
                        plsc.load_gather(ib, [skews[i] + d0, lcols])
                        for i in range(16)
                    ]
                    for i in range(16):
                        plsc.store_scatter(
                            ob, [srows, skews[i] + c0s], vs[i]
                        )
                    return carry

                lax.fori_loop(0, 32, bbody, 0)
                write(k, slot)

        def pair(kk, carry):
            step(2 * kk, 0)
            step(2 * kk + 1, 1)
            return carry

        lax.fori_loop(0, (K1_ITERS + 1) // 2, pair, 0)

        @pl.when(valid(K1_ITERS - 2))
        def _():
            wait_write(K1_ITERS - 2, (K1_ITERS - 2) % 2)

        @pl.when(valid(K1_ITERS - 1))
        def _():
            wait_write(K1_ITERS - 1, (K1_ITERS - 1) % 2)

        @pl.when(jnp.logical_and(jnp.logical_not(valid(K1_ITERS - 1)),
                                 valid(K1_ITERS - 3)))
        def _():
            wait_write(K1_ITERS - 3, (K1_ITERS - 3) % 2)

        @pl.when(wid == 0)
        def _():
            pltpu.sync_copy(tt.at[:, pl.ds(FULL_BLOCKS * 128, TAIL_COLS)], tib)

            def tqbody(q, carry):
                for g in range(4):
                    col = jnp.full((LANES,), q, jnp.int32)
                    v = plsc.load_gather(tib, [iota + 16 * g, col])
                    tob[q, pl.ds(16 * g, 16)] = v
                return carry

            lax.fori_loop(0, 64, tqbody, 0)
            pltpu.sync_copy(tob, tp.at[pl.ds(FULL_BLOCKS * 64, 64)])

    return k1


def _build_kg():
    info = plsc.get_sparse_core_info()
    nc = info.num_cores
    mesh = plsc.VectorSubcoreMesh(core_axis_name="c", subcore_axis_name="s")
    GB = 4
    OB = 2

    scratch = (
        [pltpu.VMEM((NS, 128), jnp.int32)]
        + [pltpu.VMEM((GB, 128), jnp.int32)]
        + [pltpu.VMEM((128,), jnp.int32)]
        + [pltpu.VMEM((128, 128), jnp.float32) for _ in range(GB)]
        + [pltpu.VMEM((64, 128), jnp.float32) for _ in range(OB)]
        + [pltpu.SemaphoreType.DMA for _ in range(GB)]
        + [pltpu.SemaphoreType.DMA for _ in range(OB)]
    )

    @functools.partial(
        pl.kernel,
        mesh=mesh,
        compiler_params=pltpu.CompilerParams(use_tc_tiling_on_sc=True, needs_layout_passes=False, disable_bounds_checks=True),
        out_type=jax.ShapeDtypeStruct((NS, D, NB), jnp.float32),
        scratch_types=scratch,
    )
    def kg(tp, it, out, iv, pidx, hbv, *rest):
        gbufs = rest[:GB]
        obufs = rest[GB:GB + OB]
        gsems = rest[GB + OB:GB + OB + GB]
        wsems = rest[GB + OB + GB:]
        wid = lax.axis_index("s") * nc + lax.axis_index("c")
        b0 = wid * 128
        iota = _iota16()
        skews = [jnp.bitwise_and(iota + kk2, 15) for kk2 in range(16)]

        pltpu.sync_copy(it.at[:, pl.ds(b0, 128)], iv)

        def issue_gather(s, slot):
            for rg in range(8):
                vals = iv[s, pl.ds(16 * rg, 16)]
                pidx[slot, pl.ds(16 * rg, 16)] = (
                    jnp.left_shift(jnp.right_shift(vals, 7), 6)
                    + jnp.bitwise_and(vals, 63)
                )
            return pltpu.async_copy(
                tp.at[pidx.at[slot]], gbufs[slot], gsems[slot]
            )

        def _wargs(s, slot):
            return (obufs[slot], out.at[s, :, pl.ds(b0, 128)], wsems[slot])

        def issue_write(s, slot):
            pltpu.async_copy(*_wargs(s, slot))

        def wait_write(s, slot):
            pltpu.make_async_copy(*_wargs(s, slot)).wait()

        for b in range(GB):
            issue_gather(b, b)

        def step(s, slot):
            pltpu.make_async_copy(
                tp.at[pidx.at[slot]], gbufs[slot], gsems[slot]
            ).wait()

            oslot = slot % OB

            @pl.when(s >= OB)
            def _():
                wait_write(s - OB, oslot)

            gb = gbufs[slot]
            ob = obufs[oslot]
            for rg in range(8):
                vals = iv[s, pl.ds(16 * rg, 16)]
                hbv[pl.ds(16 * rg, 16)] = jnp.bitwise_and(vals, 64)

            def rgbody(rg, carry):
                hsub = hbv[pl.ds(rg * 16, 16)]
                rvec = iota + rg * 16
                for d0 in range(0, D, 16):
                    hd = hsub + d0
                    for g0 in range(0, 16, 8):
                        vs = [
                            plsc.load_gather(gb, [rvec, hd + skews[g0 + i]])
                            for i in range(8)
                        ]
                        for i in range(8):
                            plsc.store_scatter(
                                ob, [skews[g0 + i] + d0, rvec], vs[i]
                            )
                return carry

            lax.fori_loop(0, 8, rgbody, 0)
            issue_write(s, oslot)

            @pl.when(s + GB < NS)
            def _():
                issue_gather(s + GB, slot)

        def group(ss, carry):
            for b in range(GB):
                step(ss * GB + b, b)
            return carry

        lax.fori_loop(0, NS // GB, group, 0)

        wait_write(NS - 2, (NS - 2) % GB % OB)
        wait_write(NS - 1, (NS - 1) % GB % OB)

    return kg


def kernel(level_idx, embedding_table):
    tT = embedding_table.T
    idxT = level_idx.astype(jnp.int32).T
    tpair = _build_k1()(tT)
    out3 = _build_kg()(tpair, idxT)
    return out3.transpose(2, 0, 1)


# speedup vs baseline: 1.9443x; 1.0094x over previous
import functools

import jax
import jax.numpy as jnp
from jax import lax
from jax.experimental import pallas as pl
from jax.experimental.pallas import tpu as pltpu
from jax.experimental.pallas import tpu_sc as plsc

V = 1000000
D = 64
NB = 4096
NS = 200
LANES = 16

FULL_BLOCKS = V // 128
TAIL_COLS = V - FULL_BLOCKS * 128
K1_ITERS = (FULL_BLOCKS + 31) // 32
TP_ROWS = (FULL_BLOCKS + 1) * 64


def _iota16():
    return lax.iota(jnp.int32, LANES)


def _build_k1():
    info = plsc.get_sparse_core_info()
    nc = info.num_cores
    mesh = plsc.VectorSubcoreMesh(core_axis_name="c", subcore_axis_name="s")

    scratch = [
        pltpu.VMEM((64, 128), jnp.float32),
        pltpu.VMEM((64, 128), jnp.float32),
        pltpu.VMEM((64, 128), jnp.float32),
        pltpu.VMEM((64, 128), jnp.float32),
        pltpu.VMEM((64, 64), jnp.float32),
        pltpu.VMEM((64, 128), jnp.float32),
        pltpu.SemaphoreType.DMA,
        pltpu.SemaphoreType.DMA,
        pltpu.SemaphoreType.DMA,
        pltpu.SemaphoreType.DMA,
    ]

    @functools.partial(
        pl.kernel,
        mesh=mesh,
        compiler_params=pltpu.CompilerParams(use_tc_tiling_on_sc=True, needs_layout_passes=False, disable_bounds_checks=True),
        out_type=jax.ShapeDtypeStruct((TP_ROWS, 128), jnp.float32),
        scratch_types=scratch,
    )
    def k1(tt, tp, ib0, ib1, ob0, ob1, tib, tob, rs0, rs1, ws0, ws1):
        wid = lax.axis_index("s") * nc + lax.axis_index("c")
        ibufs = (ib0, ib1)
        obufs = (ob0, ob1)
        rsems = (rs0, rs1)
        wsems = (ws0, ws1)
        iota = _iota16()
        skews = [jnp.bitwise_and(iota + kk2, 15) for kk2 in range(16)]

        def valid(k):
            return (wid + 32 * k) < FULL_BLOCKS

        def _rargs(k, slot):
            j = wid + 32 * k
            return (tt.at[:, pl.ds(j * 128, 128)], ibufs[slot], rsems[slot])

        def _wargs(k, slot):
            j = wid + 32 * k
            return (obufs[slot], tp.at[pl.ds(j * 64, 64)], wsems[slot])

        def read(k, slot):
            pltpu.async_copy(*_rargs(k, slot))

        def wait_read(k, slot):
            pltpu.make_async_copy(*_rargs(k, slot)).wait()

        def write(k, slot):
            pltpu.async_copy(*_wargs(k, slot))

        def wait_write(k, slot):
            pltpu.make_async_copy(*_wargs(k, slot)).wait()

        @pl.when(valid(0))
        def _():
            read(0, 0)

        def step(k, slot):
            @pl.when(valid(k))
            def _():
                @pl.when(valid(k + 1))
                def _():
                    read(k + 1, 1 - slot)

                wait_read(k, slot)

                @pl.when(k >= 2)
                def _():
                    wait_write(k - 2, slot)

                ib = ibufs[slot]
                ob = obufs[slot]

                def bbody(bb, carry):
                    q0s = jnp.bitwise_and(bb, 3) * 16
                    c0s = jnp.right_shift(bb, 2) * 16
                    h64 = jnp.bitwise_and(c0s, 64)
                    d0 = jnp.bitwise_and(c0s, 63)
                    lcols = iota + (h64 + q0s)
                    srows = iota + q0s
                    vs = [
                        plsc.load_gather(ib, [skews[i] + d0, lcols])
                        for i in range(16)
                    ]
                    for i in range(16):
                        plsc.store_scatter(
                            ob, [srows, skews[i] + c0s], vs[i]
                        )
                    return carry

                lax.fori_loop(0, 32, bbody, 0)
                write(k, slot)

        def pair(kk, carry):
            step(2 * kk, 0)
            step(2 * kk + 1, 1)
            return carry

        lax.fori_loop(0, (K1_ITERS + 1) // 2, pair, 0)

        @pl.when(valid(K1_ITERS - 2))
        def _():
            wait_write(K1_ITERS - 2, (K1_ITERS - 2) % 2)

        @pl.when(valid(K1_ITERS - 1))
        def _():
            wait_write(K1_ITERS - 1, (K1_ITERS - 1) % 2)

        @pl.when(jnp.logical_and(jnp.logical_not(valid(K1_ITERS - 1)),
                                 valid(K1_ITERS - 3)))
        def _():
            wait_write(K1_ITERS - 3, (K1_ITERS - 3) % 2)

        @pl.when(wid == 0)
        def _():
            pltpu.sync_copy(tt.at[:, pl.ds(FULL_BLOCKS * 128, TAIL_COLS)], tib)

            def tqbody(q, carry):
                for g in range(4):
                    col = jnp.full((LANES,), q, jnp.int32)
                    v = plsc.load_gather(tib, [iota + 16 * g, col])
                    tob[q, pl.ds(16 * g, 16)] = v
                return carry

            lax.fori_loop(0, 64, tqbody, 0)
            pltpu.sync_copy(tob, tp.at[pl.ds(FULL_BLOCKS * 64, 64)])

    return k1


def _build_kg():
    info = plsc.get_sparse_core_info()
    nc = info.num_cores
    mesh = plsc.VectorSubcoreMesh(core_axis_name="c", subcore_axis_name="s")
    GB = 4
    OB = 2

    scratch = (
        [pltpu.VMEM((NS, 128), jnp.int32)]
        + [pltpu.VMEM((GB, 128), jnp.int32)]
        + [pltpu.VMEM((128,), jnp.int32)]
        + [pltpu.VMEM((128, 128), jnp.float32) for _ in range(GB)]
        + [pltpu.VMEM((64, 128), jnp.float32) for _ in range(OB)]
        + [pltpu.SemaphoreType.DMA for _ in range(GB)]
        + [pltpu.SemaphoreType.DMA for _ in range(OB)]
    )

    @functools.partial(
        pl.kernel,
        mesh=mesh,
        compiler_params=pltpu.CompilerParams(use_tc_tiling_on_sc=True, needs_layout_passes=False, disable_bounds_checks=True),
        out_type=jax.ShapeDtypeStruct((NS, D, NB), jnp.float32),
        scratch_types=scratch,
    )
    def kg(tp, it, out, iv, pidx, hbv, *rest):
        gbufs = rest[:GB]
        obufs = rest[GB:GB + OB]
        gsems = rest[GB + OB:GB + OB + GB]
        wsems = rest[GB + OB + GB:]
        wid = lax.axis_index("s") * nc + lax.axis_index("c")
        b0 = wid * 128
        iota = _iota16()
        skews = [jnp.bitwise_and(iota + kk2, 15) for kk2 in range(16)]

        pltpu.sync_copy(it.at[:, pl.ds(b0, 128)], iv)

        def issue_gather(s, slot):
            for rg in range(8):
                vals = iv[s, pl.ds(16 * rg, 16)]
                pidx[slot, pl.ds(16 * rg, 16)] = (
                    jnp.left_shift(jnp.right_shift(vals, 7), 6)
                    + jnp.bitwise_and(vals, 63)
                )
            return pltpu.async_copy(
                tp.at[pidx.at[slot]], gbufs[slot], gsems[slot]
            )

        def _wargs(s, slot):
            return (obufs[slot], out.at[s, :, pl.ds(b0, 128)], wsems[slot])

        def issue_write(s, slot):
            pltpu.async_copy(*_wargs(s, slot))

        def wait_write(s, slot):
            pltpu.make_async_copy(*_wargs(s, slot)).wait()

        for b in range(GB):
            issue_gather(b, b)

        def step(s, slot):
            pltpu.make_async_copy(
                tp.at[pidx.at[slot]], gbufs[slot], gsems[slot]
            ).wait()

            oslot = slot % OB

            @pl.when(s >= OB)
            def _():
                wait_write(s - OB, oslot)

            gb = gbufs[slot]
            ob = obufs[oslot]
            for rg in range(8):
                vals = iv[s, pl.ds(16 * rg, 16)]
                hbv[pl.ds(16 * rg, 16)] = jnp.bitwise_and(vals, 64)

            def rgbody(rg, carry):
                hsub = hbv[pl.ds(rg * 16, 16)]
                rvec = iota + rg * 16
                for d0 in range(0, D, 16):
                    hd = hsub + d0
                    vs = [
                        plsc.load_gather(gb, [rvec, hd + skews[i]])
                        for i in range(16)
                    ]
                    for i in range(16):
                        plsc.store_scatter(
                            ob, [skews[i] + d0, rvec], vs[i]
                        )
                return carry

            lax.fori_loop(0, 8, rgbody, 0)
            issue_write(s, oslot)

            @pl.when(s + GB < NS)
            def _():
                issue_gather(s + GB, slot)

        def group(ss, carry):
            for b in range(GB):
                step(ss * GB + b, b)
            return carry

        lax.fori_loop(0, NS // GB, group, 0)

        wait_write(NS - 2, (NS - 2) % GB % OB)
        wait_write(NS - 1, (NS - 1) % GB % OB)

    return kg


def kernel(level_idx, embedding_table):
    tT = embedding_table.T
    idxT = level_idx.astype(jnp.int32).T
    tpair = _build_k1()(tT)
    out3 = _build_kg()(tpair, idxT)
    return out3.transpose(2, 0, 1)


# speedup vs baseline: 1.9487x; 1.0023x over previous
import functools

import jax
import jax.numpy as jnp
from jax import lax
from jax.experimental import pallas as pl
from jax.experimental.pallas import tpu as pltpu
from jax.experimental.pallas import tpu_sc as plsc

V = 1000000
D = 64
NB = 4096
NS = 200
LANES = 16

FULL_BLOCKS = V // 128
TAIL_COLS = V - FULL_BLOCKS * 128
K1_ITERS = (FULL_BLOCKS + 31) // 32
TP_ROWS = (FULL_BLOCKS + 1) * 64


def _iota16():
    return lax.iota(jnp.int32, LANES)


def _build_k1():
    info = plsc.get_sparse_core_info()
    nc = info.num_cores
    mesh = plsc.VectorSubcoreMesh(core_axis_name="c", subcore_axis_name="s")

    scratch = [
        pltpu.VMEM((64, 128), jnp.float32),
        pltpu.VMEM((64, 128), jnp.float32),
        pltpu.VMEM((64, 128), jnp.float32),
        pltpu.VMEM((64, 128), jnp.float32),
        pltpu.VMEM((64, 64), jnp.float32),
        pltpu.VMEM((64, 128), jnp.float32),
        pltpu.SemaphoreType.DMA,
        pltpu.SemaphoreType.DMA,
        pltpu.SemaphoreType.DMA,
        pltpu.SemaphoreType.DMA,
    ]

    @functools.partial(
        pl.kernel,
        mesh=mesh,
        compiler_params=pltpu.CompilerParams(use_tc_tiling_on_sc=True, needs_layout_passes=False, disable_bounds_checks=True),
        out_type=jax.ShapeDtypeStruct((TP_ROWS, 128), jnp.float32),
        scratch_types=scratch,
    )
    def k1(tt, tp, ib0, ib1, ob0, ob1, tib, tob, rs0, rs1, ws0, ws1):
        wid = lax.axis_index("s") * nc + lax.axis_index("c")
        ibufs = (ib0, ib1)
        obufs = (ob0, ob1)
        rsems = (rs0, rs1)
        wsems = (ws0, ws1)
        iota = _iota16()
        skews = [jnp.bitwise_and(iota + kk2, 15) for kk2 in range(16)]
        skews128 = [jnp.left_shift(sk, 7) for sk in skews]
        zero16 = iota * 0

        def valid(k):
            return (wid + 32 * k) < FULL_BLOCKS

        def _rargs(k, slot):
            j = wid + 32 * k
            return (tt.at[:, pl.ds(j * 128, 128)], ibufs[slot], rsems[slot])

        def _wargs(k, slot):
            j = wid + 32 * k
            return (obufs[slot], tp.at[pl.ds(j * 64, 64)], wsems[slot])

        def read(k, slot):
            pltpu.async_copy(*_rargs(k, slot))

        def wait_read(k, slot):
            pltpu.make_async_copy(*_rargs(k, slot)).wait()

        def write(k, slot):
            pltpu.async_copy(*_wargs(k, slot))

        def wait_write(k, slot):
            pltpu.make_async_copy(*_wargs(k, slot)).wait()

        @pl.when(valid(0))
        def _():
            read(0, 0)

        def step(k, slot):
            @pl.when(valid(k))
            def _():
                @pl.when(valid(k + 1))
                def _():
                    read(k + 1, 1 - slot)

                wait_read(k, slot)

                @pl.when(k >= 2)
                def _():
                    wait_write(k - 2, slot)

                ib = ibufs[slot]
                ob = obufs[slot]

                def bbody(bb, carry):
                    q0s = jnp.bitwise_and(bb, 3) * 16
                    c0s = jnp.right_shift(bb, 2) * 16
                    h64 = jnp.bitwise_and(c0s, 64)
                    d0 = jnp.bitwise_and(c0s, 63)
                    lflat = iota + (d0 * 128 + h64 + q0s)
                    srows = iota + q0s
                    vs = [
                        plsc.load_gather(ib, [zero16, skews128[i] + lflat])
                        for i in range(16)
                    ]
                    for i in range(16):
                        plsc.store_scatter(
                            ob, [srows, skews[i] + c0s], vs[i]
                        )
                    return carry

                lax.fori_loop(0, 32, bbody, 0)
                write(k, slot)

        def pair(kk, carry):
            step(2 * kk, 0)
            step(2 * kk + 1, 1)
            return carry

        lax.fori_loop(0, (K1_ITERS + 1) // 2, pair, 0)

        @pl.when(valid(K1_ITERS - 2))
        def _():
            wait_write(K1_ITERS - 2, (K1_ITERS - 2) % 2)

        @pl.when(valid(K1_ITERS - 1))
        def _():
            wait_write(K1_ITERS - 1, (K1_ITERS - 1) % 2)

        @pl.when(jnp.logical_and(jnp.logical_not(valid(K1_ITERS - 1)),
                                 valid(K1_ITERS - 3)))
        def _():
            wait_write(K1_ITERS - 3, (K1_ITERS - 3) % 2)

        @pl.when(wid == 0)
        def _():
            pltpu.sync_copy(tt.at[:, pl.ds(FULL_BLOCKS * 128, TAIL_COLS)], tib)

            def tqbody(q, carry):
                for g in range(4):
                    col = jnp.full((LANES,), q, jnp.int32)
                    v = plsc.load_gather(tib, [iota + 16 * g, col])
                    tob[q, pl.ds(16 * g, 16)] = v
                return carry

            lax.fori_loop(0, 64, tqbody, 0)
            pltpu.sync_copy(tob, tp.at[pl.ds(FULL_BLOCKS * 64, 64)])

    return k1


def _build_kg():
    info = plsc.get_sparse_core_info()
    nc = info.num_cores
    mesh = plsc.VectorSubcoreMesh(core_axis_name="c", subcore_axis_name="s")
    GB = 4
    OB = 2

    scratch = (
        [pltpu.VMEM((NS, 128), jnp.int32)]
        + [pltpu.VMEM((GB, 128), jnp.int32)]
        + [pltpu.VMEM((128,), jnp.int32)]
        + [pltpu.VMEM((128, 128), jnp.float32) for _ in range(GB)]
        + [pltpu.VMEM((64, 128), jnp.float32) for _ in range(OB)]
        + [pltpu.SemaphoreType.DMA for _ in range(GB)]
        + [pltpu.SemaphoreType.DMA for _ in range(OB)]
    )

    @functools.partial(
        pl.kernel,
        mesh=mesh,
        compiler_params=pltpu.CompilerParams(use_tc_tiling_on_sc=True, needs_layout_passes=False, disable_bounds_checks=True),
        out_type=jax.ShapeDtypeStruct((NS, D, NB), jnp.float32),
        scratch_types=scratch,
    )
    def kg(tp, it, out, iv, pidx, hbv, *rest):
        gbufs = rest[:GB]
        obufs = rest[GB:GB + OB]
        gsems = rest[GB + OB:GB + OB + GB]
        wsems = rest[GB + OB + GB:]
        wid = lax.axis_index("s") * nc + lax.axis_index("c")
        b0 = wid * 128
        iota = _iota16()
        skews = [jnp.bitwise_and(iota + kk2, 15) for kk2 in range(16)]

        pltpu.sync_copy(it.at[:, pl.ds(b0, 128)], iv)

        def issue_gather(s, slot):
            for rg in range(8):
                vals = iv[s, pl.ds(16 * rg, 16)]
                pidx[slot, pl.ds(16 * rg, 16)] = (
                    jnp.left_shift(jnp.right_shift(vals, 7), 6)
                    + jnp.bitwise_and(vals, 63)
                )
            return pltpu.async_copy(
                tp.at[pidx.at[slot]], gbufs[slot], gsems[slot]
            )

        def _wargs(s, slot):
            return (obufs[slot], out.at[s, :, pl.ds(b0, 128)], wsems[slot])

        def issue_write(s, slot):
            pltpu.async_copy(*_wargs(s, slot))

        def wait_write(s, slot):
            pltpu.make_async_copy(*_wargs(s, slot)).wait()

        for b in range(GB):
            issue_gather(b, b)

        def step(s, slot):
            pltpu.make_async_copy(
                tp.at[pidx.at[slot]], gbufs[slot], gsems[slot]
            ).wait()

            oslot = slot % OB

            @pl.when(s >= OB)
            def _():
                wait_write(s - OB, oslot)

            gb = gbufs[slot]
            ob = obufs[oslot]
            for rg in range(8):
                vals = iv[s, pl.ds(16 * rg, 16)]
                hbv[pl.ds(16 * rg, 16)] = jnp.bitwise_and(vals, 64)

            def rgbody(rg, carry):
                hsub = hbv[pl.ds(rg * 16, 16)]
                rvec = iota + rg * 16
                for d0 in range(0, D, 16):
                    hd = hsub + d0
                    vs = [
                        plsc.load_gather(gb, [rvec, hd + skews[i]])
                        for i in range(16)
                    ]
                    for i in range(16):
                        plsc.store_scatter(
                            ob, [skews[i] + d0, rvec], vs[i]
                        )
                return carry

            lax.fori_loop(0, 8, rgbody, 0)
            issue_write(s, oslot)

            @pl.when(s + GB < NS)
            def _():
                issue_gather(s + GB, slot)

        def group(ss, carry):
            for b in range(GB):
                step(ss * GB + b, b)
            return carry

        lax.fori_loop(0, NS // GB, group, 0)

        wait_write(NS - 2, (NS - 2) % GB % OB)
        wait_write(NS - 1, (NS - 1) % GB % OB)

    return kg


def kernel(level_idx, embedding_table):
    tT = embedding_table.T
    idxT = level_idx.astype(jnp.int32).T
    tpair = _build_k1()(tT)
    out3 = _build_kg()(tpair, idxT)
    return out3.transpose(2, 0, 1)
